# baseline (speedup 1.0000x reference)
import functools

import jax
import jax.numpy as jnp
from jax import lax
from jax.experimental import pallas as pl
from jax.experimental.pallas import tpu as pltpu
from jax.experimental.pallas import tpu_sc as plsc

_BLOCK_B = 2048
_SPLIT_FRAC = 0.5

_SC_INFO = plsc.get_sparse_core_info()
_NW = _SC_INFO.num_cores * _SC_INFO.num_subcores


@functools.lru_cache(maxsize=None)
def _make_sc_gather(V, D, B):
  b_per_w = B // _NW
  mesh = plsc.VectorSubcoreMesh(core_axis_name="c", subcore_axis_name="s")

  @functools.partial(
      pl.kernel,
      mesh=mesh,
      out_type=jax.ShapeDtypeStruct((B, D), jnp.float32),
      scratch_types=[
          pltpu.VMEM((b_per_w,), jnp.int32),
          pltpu.VMEM((b_per_w, D), jnp.float32),
          pltpu.SemaphoreType.DMA,
      ],
      name="sc_embedding_gather",
  )
  def gather_kernel(table_hbm, idx_hbm, out_hbm, idx_v, rows_v, sem):
    wid = lax.axis_index("s") * _SC_INFO.num_cores + lax.axis_index("c")
    base = wid * b_per_w
    pltpu.sync_copy(idx_hbm.at[pl.ds(base, b_per_w)], idx_v)
    pltpu.async_copy(table_hbm.at[idx_v], rows_v, sem).wait()
    pltpu.sync_copy(rows_v, out_hbm.at[pl.ds(base, b_per_w)])

  return gather_kernel


def _proj_body(feat_ref, wf_ref, bf_ref, fr_ref):
  fr_ref[...] = (jnp.dot(feat_ref[...], wf_ref[...],
                         preferred_element_type=jnp.float32) + bf_ref[...])


def _tc_proj(features, W_feat, b_feat, rows, block_b):
  B, F = features.shape
  D = W_feat.shape[1]
  return pl.pallas_call(
      _proj_body,
      grid=(rows // block_b,),
      in_specs=[
          pl.BlockSpec((block_b, F), lambda i: (i, 0)),
          pl.BlockSpec((F, D), lambda i: (0, 0)),
          pl.BlockSpec((1, D), lambda i: (0, 0)),
      ],
      out_specs=pl.BlockSpec((block_b, D), lambda i: (i, 0)),
      out_shape=jax.ShapeDtypeStruct((rows, D), jnp.float32),
  )(features, W_feat, b_feat)


def _gate_mix(idr, fr, w1a, w1b, b1, w2, b2):
  h = jnp.dot(idr, w1a, preferred_element_type=jnp.float32)
  h += jnp.dot(fr, w1b, preferred_element_type=jnp.float32)
  h = jnp.maximum(h + b1, 0.0)
  g = jnp.dot(h, w2, preferred_element_type=jnp.float32) + b2
  gate = jax.nn.sigmoid(g)
  return gate * idr + (1.0 - gate) * fr


def _gate_body(id_ref, fr_ref, w1a_ref, w1b_ref, b1_ref, w2_ref, b2_ref,
               out_ref):
  out_ref[...] = _gate_mix(id_ref[...], fr_ref[...], w1a_ref[...],
                           w1b_ref[...], b1_ref[...], w2_ref[...],
                           b2_ref[...])


def _tc_gate(id_repr, fr, W1, b1, W2, b2, rows, block_b):
  B, D = id_repr.shape
  H = W1.shape[1]
  return pl.pallas_call(
      _gate_body,
      grid=(rows // block_b,),
      in_specs=[
          pl.BlockSpec((block_b, D), lambda i: (i, 0)),
          pl.BlockSpec((block_b, D), lambda i: (i, 0)),
          pl.BlockSpec((D, H), lambda i: (0, 0)),
          pl.BlockSpec((D, H), lambda i: (1, 0)),
          pl.BlockSpec((1, H), lambda i: (0, 0)),
          pl.BlockSpec((H, D), lambda i: (0, 0)),
          pl.BlockSpec((1, D), lambda i: (0, 0)),
      ],
      out_specs=pl.BlockSpec((block_b, D), lambda i: (i, 0)),
      out_shape=jax.ShapeDtypeStruct((B, D), jnp.float32),
  )(id_repr, fr, W1, W1, b1, W2, b2)


def _fused_body(feat_ref, id_ref, wf_ref, bf_ref, w1a_ref, w1b_ref, b1_ref,
                w2_ref, b2_ref, prev_ref, out_ref):
  del prev_ref
  fr = (jnp.dot(feat_ref[...], wf_ref[...],
                preferred_element_type=jnp.float32) + bf_ref[...])
  out_ref[...] = _gate_mix(id_ref[...], fr, w1a_ref[...], w1b_ref[...],
                           b1_ref[...], w2_ref[...], b2_ref[...])


def _tc_fused_rest(features, id_repr, W_feat, b_feat, W1, b1, W2, b2, prev,
                   start_row, block_b):
  B, F = features.shape
  D = id_repr.shape[1]
  H = W1.shape[1]
  base_blk = start_row // block_b
  return pl.pallas_call(
      _fused_body,
      grid=((B - start_row) // block_b,),
      in_specs=[
          pl.BlockSpec((block_b, F), lambda i: (base_blk + i, 0)),
          pl.BlockSpec((block_b, D), lambda i: (base_blk + i, 0)),
          pl.BlockSpec((F, D), lambda i: (0, 0)),
          pl.BlockSpec((1, D), lambda i: (0, 0)),
          pl.BlockSpec((D, H), lambda i: (0, 0)),
          pl.BlockSpec((D, H), lambda i: (1, 0)),
          pl.BlockSpec((1, H), lambda i: (0, 0)),
          pl.BlockSpec((H, D), lambda i: (0, 0)),
          pl.BlockSpec((1, D), lambda i: (0, 0)),
          pl.BlockSpec(memory_space=pl.ANY),
      ],
      out_specs=pl.BlockSpec((block_b, D), lambda i: (base_blk + i, 0)),
      out_shape=jax.ShapeDtypeStruct((B, D), jnp.float32),
      input_output_aliases={9: 0},
  )(features, id_repr, W_feat, b_feat, W1, W1, b1, W2, b2, prev)


@jax.jit
def kernel(indices, features, table, W_feat, b_feat, W1, b1, W2, b2):
  V, D = table.shape
  B = indices.shape[0]
  H = W1.shape[1]
  idx = indices.astype(jnp.int32)
  bf = b_feat.reshape(1, D)
  b1r = b1.reshape(1, H)
  b2r = b2.reshape(1, D)
  rows_a = int(B * _SPLIT_FRAC) // _BLOCK_B * _BLOCK_B
  id_repr = _make_sc_gather(V, D, B)(table, idx)
  fr_a = _tc_proj(features, W_feat, bf, rows_a, _BLOCK_B)
  out_a = _tc_gate(id_repr, fr_a, W1, b1r, W2, b2r, rows_a, _BLOCK_B)
  return _tc_fused_rest(features, id_repr, W_feat, bf, W1, b1r, W2, b2r,
                        out_a, rows_a, _BLOCK_B)


# speedup vs baseline: 1.0754x; 1.0754x over previous
import functools

import jax
import jax.numpy as jnp
from jax import lax
from jax.experimental import pallas as pl
from jax.experimental.pallas import tpu as pltpu
from jax.experimental.pallas import tpu_sc as plsc

_BLOCK_B = 4096

_SC_INFO = plsc.get_sparse_core_info()
_NW = _SC_INFO.num_cores * _SC_INFO.num_subcores


@functools.lru_cache(maxsize=None)
def _make_sc_gather(V, D, B):
  b_per_w = B // _NW
  mesh = plsc.VectorSubcoreMesh(core_axis_name="c", subcore_axis_name="s")

  @functools.partial(
      pl.kernel,
      mesh=mesh,
      out_type=jax.ShapeDtypeStruct((B, D), jnp.float32),
      scratch_types=[
          pltpu.VMEM((b_per_w,), jnp.int32),
          pltpu.VMEM((b_per_w, D), jnp.float32),
          pltpu.SemaphoreType.DMA,
      ],
      name="sc_embedding_gather",
  )
  def gather_kernel(table_hbm, idx_hbm, out_hbm, idx_v, rows_v, sem):
    wid = lax.axis_index("s") * _SC_INFO.num_cores + lax.axis_index("c")
    base = wid * b_per_w
    pltpu.sync_copy(idx_hbm.at[pl.ds(base, b_per_w)], idx_v)
    pltpu.async_copy(table_hbm.at[idx_v], rows_v, sem).wait()
    pltpu.sync_copy(rows_v, out_hbm.at[pl.ds(base, b_per_w)])

  return gather_kernel


def _tc_fused_body(feat_ref, id_ref, wf_ref, bf_ref, w1a_ref, w1b_ref,
                   b1_ref, w2_ref, b2_ref, out_ref):
  idr = id_ref[...]
  fr = (jnp.dot(feat_ref[...], wf_ref[...], preferred_element_type=jnp.float32)
        + bf_ref[...])
  h = jnp.dot(idr, w1a_ref[...], preferred_element_type=jnp.float32)
  h += jnp.dot(fr, w1b_ref[...], preferred_element_type=jnp.float32)
  h = jnp.maximum(h + b1_ref[...], 0.0)
  g = jnp.dot(h, w2_ref[...], preferred_element_type=jnp.float32) + b2_ref[...]
  gate = jax.nn.sigmoid(g)
  out_ref[...] = gate * idr + (1.0 - gate) * fr


def _tc_fused(features, id_repr, W_feat, b_feat, W1, b1, W2, b2,
              block_b=_BLOCK_B):
  B, F = features.shape
  D = id_repr.shape[1]
  H = W1.shape[1]
  full = lambda *s: pl.BlockSpec(s, lambda i: (0,) * len(s))
  return pl.pallas_call(
      _tc_fused_body,
      grid=(B // block_b,),
      in_specs=[
          pl.BlockSpec((block_b, F), lambda i: (i, 0)),
          pl.BlockSpec((block_b, D), lambda i: (i, 0)),
          full(F, D),
          pl.BlockSpec((1, D), lambda i: (0, 0)),
          pl.BlockSpec((D, H), lambda i: (0, 0)),
          pl.BlockSpec((D, H), lambda i: (1, 0)),
          pl.BlockSpec((1, H), lambda i: (0, 0)),
          full(H, D),
          pl.BlockSpec((1, D), lambda i: (0, 0)),
      ],
      out_specs=pl.BlockSpec((block_b, D), lambda i: (i, 0)),
      out_shape=jax.ShapeDtypeStruct((B, D), jnp.float32),
  )(features, id_repr, W_feat, b_feat.reshape(1, D), W1, W1,
    b1.reshape(1, H), W2, b2.reshape(1, D))


@jax.jit
def kernel(indices, features, table, W_feat, b_feat, W1, b1, W2, b2):
  V, D = table.shape
  B = indices.shape[0]
  idx = indices.astype(jnp.int32)
  id_repr = _make_sc_gather(V, D, B)(table, idx)
  return _tc_fused(features, id_repr, W_feat, b_feat, W1, b1, W2, b2)


# speedup vs baseline: 1.0777x; 1.0021x over previous
import functools

import jax
import jax.numpy as jnp
from jax import lax
from jax.experimental import pallas as pl
from jax.experimental.pallas import tpu as pltpu
from jax.experimental.pallas import tpu_sc as plsc

_BLOCK_B = 4096

_SC_INFO = plsc.get_sparse_core_info()
_NW = _SC_INFO.num_cores * _SC_INFO.num_subcores


@functools.lru_cache(maxsize=None)
def _make_sc_gather(V, D, B):
  b_per_w = B // _NW
  mesh = plsc.VectorSubcoreMesh(core_axis_name="c", subcore_axis_name="s")

  @functools.partial(
      pl.kernel,
      mesh=mesh,
      out_type=jax.ShapeDtypeStruct((B, D), jnp.float32),
      scratch_types=[
          pltpu.VMEM((b_per_w,), jnp.int32),
          pltpu.VMEM((b_per_w, D), jnp.float32),
          pltpu.SemaphoreType.DMA,
      ],
      name="sc_embedding_gather",
  )
  def gather_kernel(table_hbm, idx_hbm, out_hbm, idx_v, rows_v, sem):
    wid = lax.axis_index("s") * _SC_INFO.num_cores + lax.axis_index("c")
    base = wid * b_per_w
    pltpu.sync_copy(idx_hbm.at[pl.ds(base, b_per_w)], idx_v)
    pltpu.async_copy(table_hbm.at[idx_v], rows_v, sem).wait()
    pltpu.sync_copy(rows_v, out_hbm.at[pl.ds(base, b_per_w)])

  return gather_kernel


def _tc_fused_body(feat_ref, id_ref, wf_ref, bf_ref, w1a_ref, w1b_ref,
                   b1_ref, w2_ref, b2_ref, out_ref):
  idr = id_ref[...]
  fr = (jnp.dot(feat_ref[...], wf_ref[...], preferred_element_type=jnp.float32)
        + bf_ref[...])
  h = jnp.dot(idr, w1a_ref[...], preferred_element_type=jnp.float32)
  h += jnp.dot(fr, w1b_ref[...], preferred_element_type=jnp.float32)
  h = jnp.maximum(h + b1_ref[...], 0.0)
  g = jnp.dot(h, w2_ref[...], preferred_element_type=jnp.float32) + b2_ref[...]
  gate = jax.nn.sigmoid(g)
  out_ref[...] = gate * idr + (1.0 - gate) * fr


def _tc_fused(features, id_repr, W_feat, b_feat, W1, b1, W2, b2,
              block_b=_BLOCK_B):
  B, F = features.shape
  D = id_repr.shape[1]
  H = W1.shape[1]
  full = lambda *s: pl.BlockSpec(s, lambda i: (0,) * len(s))
  return pl.pallas_call(
      _tc_fused_body,
      grid=(B // block_b,),
      in_specs=[
          pl.BlockSpec((block_b, F), lambda i: (i, 0)),
          pl.BlockSpec((block_b, D), lambda i: (i, 0)),
          full(F, D),
          pl.BlockSpec((1, D), lambda i: (0, 0)),
          pl.BlockSpec((D, H), lambda i: (0, 0)),
          pl.BlockSpec((D, H), lambda i: (1, 0)),
          pl.BlockSpec((1, H), lambda i: (0, 0)),
          full(H, D),
          pl.BlockSpec((1, D), lambda i: (0, 0)),
      ],
      out_specs=pl.BlockSpec((block_b, D), lambda i: (i, 0)),
      out_shape=jax.ShapeDtypeStruct((B, D), jnp.float32),
      input_output_aliases={1: 0},
  )(features, id_repr, W_feat, b_feat.reshape(1, D), W1, W1,
    b1.reshape(1, H), W2, b2.reshape(1, D))


@jax.jit
def kernel(indices, features, table, W_feat, b_feat, W1, b1, W2, b2):
  V, D = table.shape
  B = indices.shape[0]
  idx = indices.astype(jnp.int32)
  id_repr = _make_sc_gather(V, D, B)(table, idx)
  return _tc_fused(features, id_repr, W_feat, b_feat, W1, b1, W2, b2)
